# baseline (speedup 1.0000x reference)
import functools

import jax
import jax.numpy as jnp
from jax.experimental import pallas as pl
from jax.experimental.pallas import tpu as pltpu

_BLOCK_S = 512


def _gelu_tile_kernel(table_ref, out_ref, *, batch: int):
    x = table_ref[...]
    y = 0.5 * x * (1.0 + jax.lax.erf(x * (2.0 ** -0.5)))
    out_ref[...] = jnp.concatenate([y] * batch, axis=1)


def kernel(seq, emb_table):
    seq_len, batch = seq.shape
    hidden = emb_table.shape[1]
    grid = seq_len // _BLOCK_S

    out2d = pl.pallas_call(
        functools.partial(_gelu_tile_kernel, batch=batch),
        grid=(grid,),
        in_specs=[pl.BlockSpec((_BLOCK_S, hidden), lambda i: (i, 0))],
        out_specs=pl.BlockSpec((_BLOCK_S, batch * hidden), lambda i: (i, 0)),
        out_shape=jax.ShapeDtypeStruct((seq_len, batch * hidden), emb_table.dtype),
        compiler_params=pltpu.CompilerParams(
            dimension_semantics=("parallel",),
        ),
    )(emb_table)
    return out2d.reshape(seq_len, batch, hidden)
